# baseline (speedup 1.0000x reference)
import jax
import jax.numpy as jnp
from jax import lax
from jax.experimental import pallas as pl
from jax.experimental.pallas import tpu as pltpu
from jax.experimental.pallas import tpu_sc as plsc

_NUM_CORES = 2
_NUM_SUBCORES = 16
_NUM_WORKERS = _NUM_CORES * _NUM_SUBCORES
_CHUNK = 200
_K = 2
_WPAD = 128


def kernel(token_ids, weight):
    batch, seq = token_ids.shape
    num_indices = batch * seq
    dim = weight.shape[1]
    per_worker = num_indices // _NUM_WORKERS
    n_chunks = per_worker // _CHUNK
    n_groups = n_chunks // _K
    idx_flat = token_ids.reshape(num_indices).astype(jnp.int32)
    w128 = jnp.pad(weight, ((0, 0), (0, _WPAD - dim)))

    mesh = plsc.VectorSubcoreMesh(core_axis_name="c", subcore_axis_name="s")

    @pl.kernel(
        out_type=jax.ShapeDtypeStruct((num_indices, dim), weight.dtype),
        mesh=mesh,
        compiler_params=pltpu.CompilerParams(use_tc_tiling_on_sc=False),
        scratch_types=[
            pltpu.VMEM((per_worker,), jnp.int32),
            pltpu.VMEM((2, _K, _CHUNK, _WPAD), jnp.float32),
            pltpu.SemaphoreType.DMA,
            pltpu.SemaphoreType.DMA,
            pltpu.SemaphoreType.DMA,
            pltpu.SemaphoreType.DMA,
        ],
    )
    def gather_kernel(w_hbm, i_hbm, o_hbm, idx_v, rows, gs_a, ss_a, gs_b, ss_b):
        wid = lax.axis_index("s") * _NUM_CORES + lax.axis_index("c")
        base = wid * per_worker
        pltpu.sync_copy(i_hbm.at[pl.ds(base, per_worker)], idx_v)

        def issue_gathers(set_i, group, gsem):
            for b in range(_K):
                off = (group * _K + b) * _CHUNK
                pltpu.async_copy(
                    w_hbm.at[idx_v.at[pl.ds(off, _CHUNK)]], rows.at[set_i, b], gsem
                )

        def drain_gathers(set_i, gsem):
            for b in range(_K):
                pltpu.make_async_copy(
                    w_hbm.at[pl.ds(0, _CHUNK)], rows.at[set_i, b], gsem
                ).wait()

        def issue_stores(set_i, group, ssem):
            for b in range(_K):
                off = (group * _K + b) * _CHUNK
                pltpu.async_copy(
                    rows.at[set_i, b, :, pl.ds(0, dim)],
                    o_hbm.at[pl.ds(base + off, _CHUNK)],
                    ssem,
                )

        def drain_stores(set_i, group, ssem):
            for b in range(_K):
                off = (group * _K + b) * _CHUNK
                pltpu.make_async_copy(
                    rows.at[set_i, b, :, pl.ds(0, dim)],
                    o_hbm.at[pl.ds(base + off, _CHUNK)],
                    ssem,
                ).wait()

        issue_gathers(0, 0, gs_a)
        issue_gathers(1, 1, gs_b)

        @pl.loop(0, n_groups, step=2)
        def _(g):
            drain_gathers(0, gs_a)
            issue_stores(0, g, ss_a)
            drain_stores(0, g, ss_a)

            @pl.when(g + 2 < n_groups)
            def _():
                issue_gathers(0, g + 2, gs_a)

            drain_gathers(1, gs_b)
            issue_stores(1, g + 1, ss_b)
            drain_stores(1, g + 1, ss_b)

            @pl.when(g + 3 < n_groups)
            def _():
                issue_gathers(1, g + 3, gs_b)

    out = gather_kernel(w128, idx_flat)
    return out.reshape(batch, seq, dim)


# speedup vs baseline: 1.2473x; 1.2473x over previous
import jax
import jax.numpy as jnp
from jax import lax
from jax.experimental import pallas as pl
from jax.experimental.pallas import tpu as pltpu
from jax.experimental.pallas import tpu_sc as plsc

_NUM_CORES = 2
_NUM_SUBCORES = 16
_NUM_WORKERS = _NUM_CORES * _NUM_SUBCORES
_CHUNK = 200
_K = 2
_WPAD = 128


def kernel(token_ids, weight):
    batch, seq = token_ids.shape
    num_indices = batch * seq
    dim = weight.shape[1]
    per_worker = num_indices // _NUM_WORKERS
    n_chunks = per_worker // _CHUNK
    n_groups = n_chunks // _K
    idx_flat = token_ids.reshape(num_indices).astype(jnp.int32)
    w128 = jnp.pad(weight, ((0, 0), (0, _WPAD - dim)))

    mesh = plsc.VectorSubcoreMesh(core_axis_name="c", subcore_axis_name="s")

    @pl.kernel(
        out_type=jax.ShapeDtypeStruct((batch, 56, _WPAD), weight.dtype),
        mesh=mesh,
        compiler_params=pltpu.CompilerParams(use_tc_tiling_on_sc=False),
        scratch_types=[
            pltpu.VMEM((per_worker,), jnp.int32),
            pltpu.VMEM((2, _K, _CHUNK, _WPAD), jnp.float32),
            pltpu.SemaphoreType.DMA,
            pltpu.SemaphoreType.DMA,
            pltpu.SemaphoreType.DMA,
            pltpu.SemaphoreType.DMA,
        ],
    )
    def gather_kernel(w_hbm, i_hbm, o_hbm, idx_v, rows, gs_a, ss_a, gs_b, ss_b):
        wid = lax.axis_index("s") * _NUM_CORES + lax.axis_index("c")
        base = wid * per_worker
        pltpu.sync_copy(i_hbm.at[pl.ds(base, per_worker)], idx_v)

        def issue_gathers(set_i, group, gsem):
            for b in range(_K):
                off = (group * _K + b) * _CHUNK
                pltpu.async_copy(
                    w_hbm.at[idx_v.at[pl.ds(off, _CHUNK)]], rows.at[set_i, b], gsem
                )

        def drain_gathers(set_i, gsem):
            for b in range(_K):
                pltpu.make_async_copy(
                    w_hbm.at[pl.ds(0, _CHUNK)], rows.at[set_i, b], gsem
                ).wait()

        def issue_stores(set_i, group, ssem):
            for b in range(_K):
                off = (group * _K + b) * _CHUNK
                for j in range(_CHUNK // seq):
                    pltpu.async_copy(
                        rows.at[set_i, b, pl.ds(j * seq, seq), :],
                        o_hbm.at[(base + off) // seq + j, pl.ds(0, seq)],
                        ssem,
                    )

        def drain_stores(set_i, group, ssem):
            for b in range(_K):
                off = (group * _K + b) * _CHUNK
                for j in range(_CHUNK // seq):
                    pltpu.make_async_copy(
                        rows.at[set_i, b, pl.ds(j * seq, seq), :],
                        o_hbm.at[(base + off) // seq + j, pl.ds(0, seq)],
                        ssem,
                    ).wait()

        issue_gathers(0, 0, gs_a)
        issue_gathers(1, 1, gs_b)

        @pl.loop(0, n_groups, step=2)
        def _(g):
            drain_gathers(0, gs_a)
            issue_stores(0, g, ss_a)
            drain_stores(0, g, ss_a)

            @pl.when(g + 2 < n_groups)
            def _():
                issue_gathers(0, g + 2, gs_a)

            drain_gathers(1, gs_b)
            issue_stores(1, g + 1, ss_b)
            drain_stores(1, g + 1, ss_b)

            @pl.when(g + 3 < n_groups)
            def _():
                issue_gathers(1, g + 3, gs_b)

    out = gather_kernel(w128, idx_flat)
    return out[:, :seq, :dim]


# speedup vs baseline: 1.3487x; 1.0812x over previous
import jax
import jax.numpy as jnp
from jax import lax
from jax.experimental import pallas as pl
from jax.experimental.pallas import tpu as pltpu
from jax.experimental.pallas import tpu_sc as plsc

_NUM_CORES = 2
_NUM_SUBCORES = 16
_NUM_WORKERS = _NUM_CORES * _NUM_SUBCORES
_CHUNK = 200
_K = 2
_WPAD = 128


def kernel(token_ids, weight):
    batch, seq = token_ids.shape
    num_indices = batch * seq
    dim = weight.shape[1]
    per_worker = num_indices // _NUM_WORKERS
    n_chunks = per_worker // _CHUNK
    n_groups = n_chunks // _K
    idx_flat = token_ids.reshape(num_indices).astype(jnp.int32)
    w128 = jnp.pad(weight, ((0, 0), (0, _WPAD - dim)))

    mesh = plsc.VectorSubcoreMesh(core_axis_name="c", subcore_axis_name="s")

    @pl.kernel(
        out_type=jax.ShapeDtypeStruct((batch, 56, _WPAD), weight.dtype),
        mesh=mesh,
        compiler_params=pltpu.CompilerParams(use_tc_tiling_on_sc=False),
        scratch_types=[
            pltpu.VMEM((per_worker,), jnp.int32),
            pltpu.VMEM((2, _K, _CHUNK, _WPAD), jnp.float32),
            pltpu.SemaphoreType.DMA,
            pltpu.SemaphoreType.DMA,
            pltpu.SemaphoreType.DMA,
            pltpu.SemaphoreType.DMA,
        ],
    )
    def gather_kernel(w_hbm, i_hbm, o_hbm, idx_v, rows, gs_a, ss_a, gs_b, ss_b):
        wid = lax.axis_index("s") * _NUM_CORES + lax.axis_index("c")
        base = wid * per_worker
        pltpu.sync_copy(i_hbm.at[pl.ds(base, per_worker)], idx_v)

        def issue_gathers(set_i, group, gsem):
            for b in range(_K):
                off = (group * _K + b) * _CHUNK
                pltpu.async_copy(
                    w_hbm.at[idx_v.at[pl.ds(off, _CHUNK)]], rows.at[set_i, b], gsem
                )

        def drain_gathers(set_i, gsem):
            for b in range(_K):
                pltpu.make_async_copy(
                    w_hbm.at[pl.ds(0, _CHUNK)], rows.at[set_i, b], gsem
                ).wait()

        def issue_stores(set_i, group, ssem):
            for b in range(_K):
                off = (group * _K + b) * _CHUNK
                for j in range(_CHUNK // seq):
                    pltpu.async_copy(
                        rows.at[set_i, b, pl.ds(j * seq, seq), pl.ds(0, dim)],
                        o_hbm.at[(base + off) // seq + j, pl.ds(0, seq), pl.ds(0, dim)],
                        ssem,
                    )

        def drain_stores(set_i, group, ssem):
            for b in range(_K):
                off = (group * _K + b) * _CHUNK
                for j in range(_CHUNK // seq):
                    pltpu.make_async_copy(
                        rows.at[set_i, b, pl.ds(j * seq, seq), pl.ds(0, dim)],
                        o_hbm.at[(base + off) // seq + j, pl.ds(0, seq), pl.ds(0, dim)],
                        ssem,
                    ).wait()

        issue_gathers(0, 0, gs_a)
        issue_gathers(1, 1, gs_b)

        @pl.loop(0, n_groups, step=2)
        def _(g):
            drain_gathers(0, gs_a)
            issue_stores(0, g, ss_a)
            drain_stores(0, g, ss_a)

            @pl.when(g + 2 < n_groups)
            def _():
                issue_gathers(0, g + 2, gs_a)

            drain_gathers(1, gs_b)
            issue_stores(1, g + 1, ss_b)
            drain_stores(1, g + 1, ss_b)

            @pl.when(g + 3 < n_groups)
            def _():
                issue_gathers(1, g + 3, gs_b)

    out = gather_kernel(w128, idx_flat)
    return out[:, :seq, :dim]


# speedup vs baseline: 1.4589x; 1.0817x over previous
import jax
import jax.numpy as jnp
from jax import lax
from jax.experimental import pallas as pl
from jax.experimental.pallas import tpu as pltpu
from jax.experimental.pallas import tpu_sc as plsc

_NUM_CORES = 2
_NUM_SUBCORES = 16
_NUM_WORKERS = _NUM_CORES * _NUM_SUBCORES
_CHUNK = 200
_K = 2
_WPAD = 128


def kernel(token_ids, weight):
    batch, seq = token_ids.shape
    num_indices = batch * seq
    dim = weight.shape[1]
    per_worker = num_indices // _NUM_WORKERS
    n_chunks = per_worker // _CHUNK
    n_groups = n_chunks // _K
    idx_flat = token_ids.reshape(num_indices).astype(jnp.int32) * 2
    w128 = jnp.pad(weight, ((0, 0), (0, _WPAD - dim)))
    w2 = w128.reshape(2 * weight.shape[0], dim)

    mesh = plsc.VectorSubcoreMesh(core_axis_name="c", subcore_axis_name="s")

    @pl.kernel(
        out_type=jax.ShapeDtypeStruct((batch, 56, _WPAD), weight.dtype),
        mesh=mesh,
        compiler_params=pltpu.CompilerParams(use_tc_tiling_on_sc=False),
        scratch_types=[
            pltpu.VMEM((per_worker,), jnp.int32),
            pltpu.VMEM((2, _K, _CHUNK, dim), jnp.float32),
            pltpu.SemaphoreType.DMA,
            pltpu.SemaphoreType.DMA,
            pltpu.SemaphoreType.DMA,
            pltpu.SemaphoreType.DMA,
        ],
    )
    def gather_kernel(w_hbm, i_hbm, o_hbm, idx_v, rows, gs_a, ss_a, gs_b, ss_b):
        wid = lax.axis_index("s") * _NUM_CORES + lax.axis_index("c")
        base = wid * per_worker
        pltpu.sync_copy(i_hbm.at[pl.ds(base, per_worker)], idx_v)

        def issue_gathers(set_i, group, gsem):
            for b in range(_K):
                off = (group * _K + b) * _CHUNK
                pltpu.async_copy(
                    w_hbm.at[idx_v.at[pl.ds(off, _CHUNK)]], rows.at[set_i, b], gsem
                )

        def drain_gathers(set_i, gsem):
            for b in range(_K):
                pltpu.make_async_copy(
                    w_hbm.at[pl.ds(0, _CHUNK)], rows.at[set_i, b], gsem
                ).wait()

        def issue_stores(set_i, group, ssem):
            for b in range(_K):
                off = (group * _K + b) * _CHUNK
                for j in range(_CHUNK // seq):
                    pltpu.async_copy(
                        rows.at[set_i, b, pl.ds(j * seq, seq), :],
                        o_hbm.at[(base + off) // seq + j, pl.ds(0, seq), pl.ds(0, dim)],
                        ssem,
                    )

        def drain_stores(set_i, group, ssem):
            for b in range(_K):
                off = (group * _K + b) * _CHUNK
                for j in range(_CHUNK // seq):
                    pltpu.make_async_copy(
                        rows.at[set_i, b, pl.ds(j * seq, seq), :],
                        o_hbm.at[(base + off) // seq + j, pl.ds(0, seq), pl.ds(0, dim)],
                        ssem,
                    ).wait()

        issue_gathers(0, 0, gs_a)
        issue_gathers(1, 1, gs_b)

        @pl.loop(0, n_groups, step=2)
        def _(g):
            drain_gathers(0, gs_a)
            issue_stores(0, g, ss_a)
            drain_stores(0, g, ss_a)

            @pl.when(g + 2 < n_groups)
            def _():
                issue_gathers(0, g + 2, gs_a)

            drain_gathers(1, gs_b)
            issue_stores(1, g + 1, ss_b)
            drain_stores(1, g + 1, ss_b)

            @pl.when(g + 3 < n_groups)
            def _():
                issue_gathers(1, g + 3, gs_b)

    out = gather_kernel(w2, idx_flat)
    return out[:, :seq, :dim]
